# baseline (speedup 1.0000x reference)
import functools

import jax
import jax.numpy as jnp
from jax.experimental import pallas as pl
from jax.experimental.pallas import tpu as pltpu


def _mm_body(x_ref, w_ref, b_ref, o_ref):
    o_ref[...] = (
        jnp.dot(x_ref[...], w_ref[...], preferred_element_type=jnp.float32)
        + b_ref[...]
    )


def _matmul_bias(x, w, b, block_rows=1000):
    r, d = x.shape
    dout = w.shape[1]
    assert r % block_rows == 0, (r, block_rows)
    grid = (r // block_rows,)
    return pl.pallas_call(
        _mm_body,
        grid=grid,
        in_specs=[
            pl.BlockSpec((block_rows, d), lambda i: (i, 0)),
            pl.BlockSpec((d, dout), lambda i: (0, 0)),
            pl.BlockSpec((1, dout), lambda i: (0, 0)),
        ],
        out_specs=pl.BlockSpec((block_rows, dout), lambda i: (i, 0)),
        out_shape=jax.ShapeDtypeStruct((r, dout), jnp.float32),
    )(x, w, b.reshape(1, dout))


def _qkv(x, p):
    w = jnp.concatenate([p['Wq'], p['Wk'], p['Wv']], axis=1)
    b = jnp.concatenate([p['bq'], p['bk'], p['bv']], axis=0)
    out = _matmul_bias(x, w, b)
    d = p['Wq'].shape[1]
    return out[:, :d], out[:, d:2 * d], out[:, 2 * d:]


def _seg_softmax(logits, index, num_segments):
    m = jax.ops.segment_max(logits, index, num_segments=num_segments)
    m = jnp.where(jnp.isfinite(m), m, 0.0)
    e = jnp.exp(logits - m[index])
    s = jax.ops.segment_sum(e, index, num_segments=num_segments)
    return e / (s[index] + 1e-16)


def _sparse_attn(q, k, v, src, dst, num_dst, scale):
    logits = (q[dst] * k[src]).sum(axis=-1) * scale
    alpha = _seg_softmax(logits, dst, num_dst)
    return jax.ops.segment_sum(alpha[:, None] * v[src], dst, num_segments=num_dst)


def kernel(x_node, x_tri, params, node_edge_index, tri_edge_index,
           nt_edge_index, tn_edge_index):
    t, n, d = x_node.shape
    _, m, _ = x_tri.shape
    xn = x_node.reshape(t * n, d)
    xt = x_tri.reshape(t * m, d)
    scale = d ** (-0.5)

    ne_src = node_edge_index[0].astype(jnp.int32)
    ne_dst = node_edge_index[1].astype(jnp.int32)
    te_src = tri_edge_index[0].astype(jnp.int32)
    te_dst = tri_edge_index[1].astype(jnp.int32)
    nt_src = nt_edge_index[0].astype(jnp.int32)
    nt_dst = nt_edge_index[1].astype(jnp.int32)
    tn_src = tn_edge_index[0].astype(jnp.int32)
    tn_dst = tn_edge_index[1].astype(jnp.int32)

    p = params['nsa']
    q, k, v = _qkv(xn, p)
    h_node = _sparse_attn(q, k, v, ne_src, ne_dst, t * n, scale)

    p = params['n2t']
    q = _matmul_bias(xt, p['Wq'], p['bq'])
    kk = _matmul_bias(h_node, p['Wk'], p['bk'])
    vv = _matmul_bias(h_node, p['Wv'], p['bv'])
    aggr = _sparse_attn(q, kk, vv, nt_src, nt_dst, t * m, scale)
    h_tri = _matmul_bias(aggr, p['Wo'], p['bo']) + xt

    p = params['tsa']
    q, k, v = _qkv(h_tri, p)
    h_tri2 = _sparse_attn(q, k, v, te_src, te_dst, t * m, d ** 0.5)

    p = params['t2n']
    q = _matmul_bias(h_node, p['Wq'], p['bq'])
    kk = _matmul_bias(h_tri2, p['Wk'], p['bk'])
    vv = _matmul_bias(h_tri2, p['Wv'], p['bv'])
    aggr = _sparse_attn(q, kk, vv, tn_src, tn_dst, t * n, scale)
    out_node = _matmul_bias(aggr, p['Wo'], p['bo']) + h_node
    return out_node.reshape(t, n, d)


# speedup vs baseline: 1.3635x; 1.3635x over previous
import functools

import jax
import jax.numpy as jnp
import numpy as np
from jax import lax
from jax.experimental import pallas as pl
from jax.experimental.pallas import tpu as pltpu
from jax.experimental.pallas import tpu_sc as plsc

NC, NS, LANES = 2, 16, 16
NW = NC * NS
B = 128
D = 256
NEG = -3.0e38

_MESH = plsc.VectorSubcoreMesh(core_axis_name="c", subcore_axis_name="s")
_SC_PARAMS = pltpu.CompilerParams(needs_layout_passes=False,
                                  use_tc_tiling_on_sc=False)

def _lane_iota():
    return lax.broadcasted_iota(jnp.int32, (LANES,), 0)


def _lane_gather(vec16, idx16):
    dn = lax.GatherDimensionNumbers(offset_dims=(), collapsed_slice_dims=(0,),
                                    start_index_map=(0,))
    return lax.gather(vec16, idx16.reshape(LANES, 1), dn, slice_sizes=(1,),
                      mode=lax.GatherScatterMode.PROMISE_IN_BOUNDS)


def _bcast_lane(vec16, lane, j):
    return _lane_gather(vec16, lane * 0 + j)


def _lane_sum(acc, lane):
    for sh in (1, 2, 4, 8):
        acc = acc + _lane_gather(acc, lane ^ sh)
    return acc


def _mm_body(x_ref, w_ref, b_ref, o_ref):
    o_ref[...] = (
        jnp.dot(x_ref[...], w_ref[...], preferred_element_type=jnp.float32)
        + b_ref[...]
    )


def _mm_res_body(x_ref, w_ref, b_ref, r_ref, o_ref):
    o_ref[...] = (
        jnp.dot(x_ref[...], w_ref[...], preferred_element_type=jnp.float32)
        + b_ref[...] + r_ref[...]
    )


def _matmul_bias(x, w, b, res=None, block_rows=1024):
    r, d = x.shape
    dout = w.shape[1]
    assert r % block_rows == 0, (r, block_rows)
    grid = (r // block_rows,)
    if res is None:
        return pl.pallas_call(
            _mm_body,
            grid=grid,
            in_specs=[
                pl.BlockSpec((block_rows, d), lambda i: (i, 0)),
                pl.BlockSpec((d, dout), lambda i: (0, 0)),
                pl.BlockSpec((1, dout), lambda i: (0, 0)),
            ],
            out_specs=pl.BlockSpec((block_rows, dout), lambda i: (i, 0)),
            out_shape=jax.ShapeDtypeStruct((r, dout), jnp.float32),
        )(x, w, b.reshape(1, dout))
    return pl.pallas_call(
        _mm_res_body,
        grid=grid,
        in_specs=[
            pl.BlockSpec((block_rows, d), lambda i: (i, 0)),
            pl.BlockSpec((d, dout), lambda i: (0, 0)),
            pl.BlockSpec((1, dout), lambda i: (0, 0)),
            pl.BlockSpec((block_rows, dout), lambda i: (i, 0)),
        ],
        out_specs=pl.BlockSpec((block_rows, dout), lambda i: (i, 0)),
        out_shape=jax.ShapeDtypeStruct((r, dout), jnp.float32),
    )(x, w, b.reshape(1, dout), res)


def _qkv(x, p, scale):
    w = jnp.concatenate([p['Wq'] * scale, p['Wk'], p['Wv']], axis=1)
    b = jnp.concatenate([p['bq'] * scale, p['bk'], p['bv']], axis=0)
    out = _matmul_bias(x, w, b)
    return out[:, :D], out[:, D:2 * D], out[:, 2 * D:]


@functools.lru_cache(maxsize=None)
def _k1_logits(EP, NdP, use_max):
    chunk = EP // NW
    nblk = chunk // B

    scratch = [
        pltpu.VMEM((B,), jnp.int32),
        pltpu.VMEM((B,), jnp.int32),
        pltpu.VMEM((B, D), jnp.float32),
        pltpu.VMEM((B, D), jnp.float32),
        pltpu.VMEM((B,), jnp.float32),
    ]
    if use_max:
        scratch.append(pltpu.VMEM((NdP,), jnp.float32))
        out_type = (jax.ShapeDtypeStruct((EP,), jnp.float32),
                    jax.ShapeDtypeStruct((NW, NdP), jnp.float32))
    else:
        out_type = jax.ShapeDtypeStruct((EP,), jnp.float32)

    @functools.partial(pl.kernel, out_type=out_type, mesh=_MESH,
                       scratch_types=scratch, compiler_params=_SC_PARAMS)
    def k1(q_hbm, k_hbm, src_hbm, dst_hbm, *rest):
        if use_max:
            l_hbm, mpart_hbm, src_v, dst_v, q_v, k_v, l_v, m_v = rest
        else:
            l_hbm, src_v, dst_v, q_v, k_v, l_v = rest
        w = lax.axis_index("s") * NC + lax.axis_index("c")
        base = w * chunk

        if use_max:
            def initm(i, carry):
                m_v[pl.ds(i * LANES, LANES)] = jnp.full((LANES,), NEG,
                                                        jnp.float32)
                return carry
            lax.fori_loop(0, NdP // LANES, initm, 0)

        def blk(b, carry):
            off = base + b * B
            pltpu.sync_copy(src_hbm.at[pl.ds(off, B)], src_v)
            pltpu.sync_copy(dst_hbm.at[pl.ds(off, B)], dst_v)
            pltpu.sync_copy(q_hbm.at[dst_v], q_v)
            pltpu.sync_copy(k_hbm.at[src_v], k_v)

            def dotgrp(g, ecarry):
                lane = _lane_iota()
                lvec = jnp.zeros((LANES,), jnp.float32)
                for j in range(LANES):
                    e = g * LANES + j
                    acc = q_v[e, pl.ds(0, LANES)] * k_v[e, pl.ds(0, LANES)]
                    for c in range(1, D // LANES):
                        acc = acc + (q_v[e, pl.ds(c * LANES, LANES)]
                                     * k_v[e, pl.ds(c * LANES, LANES)])
                    lvec = jnp.where(lane == j, _lane_sum(acc, lane), lvec)
                l_v[pl.ds(g * LANES, LANES)] = lvec
                return ecarry
            lax.fori_loop(0, B // LANES, dotgrp, 0)

            if use_max:
                def grp(g, gcarry):
                    sl = pl.ds(g * LANES, LANES)
                    l16 = l_v[sl]
                    d16 = dst_v[sl]

                    def cond(c_):
                        return c_

                    def body(c_):
                        mo = plsc.load_gather(m_v, [d16])
                        plsc.store_scatter(m_v, [d16], l16, mask=l16 > mo)
                        mo2 = plsc.load_gather(m_v, [d16])
                        return jnp.any(l16 > mo2)
                    lax.while_loop(cond, body, True)
                    return gcarry
                lax.fori_loop(0, B // LANES, grp, 0)

            pltpu.sync_copy(l_v, l_hbm.at[pl.ds(off, B)])
            return carry
        lax.fori_loop(0, nblk, blk, 0)

        if use_max:
            pltpu.sync_copy(m_v, mpart_hbm.at[w])
    return k1


@functools.lru_cache(maxsize=None)
def _k2_merge(NdP):
    sl_len = NdP // NW

    @functools.partial(
        pl.kernel,
        out_type=jax.ShapeDtypeStruct((NdP,), jnp.float32),
        mesh=_MESH,
        scratch_types=[pltpu.VMEM((sl_len,), jnp.float32),
                       pltpu.VMEM((sl_len,), jnp.float32)],
        compiler_params=_SC_PARAMS)
    def k2(mpart_hbm, mfin_hbm, acc_v, tmp_v):
        w = lax.axis_index("s") * NC + lax.axis_index("c")
        off = w * sl_len
        pltpu.sync_copy(mpart_hbm.at[0, pl.ds(off, sl_len)], acc_v)

        def red(w2, carry):
            pltpu.sync_copy(mpart_hbm.at[w2, pl.ds(off, sl_len)], tmp_v)

            def ch(i, icarry):
                s_ = pl.ds(i * LANES, LANES)
                acc_v[s_] = jnp.maximum(acc_v[s_], tmp_v[s_])
                return icarry
            lax.fori_loop(0, sl_len // LANES, ch, 0)
            return carry
        lax.fori_loop(1, NW, red, 0)
        pltpu.sync_copy(acc_v, mfin_hbm.at[pl.ds(off, sl_len)])
    return k2


@functools.lru_cache(maxsize=None)
def _k3_expsum(EP, NdP, use_max):
    chunk = EP // NW
    nblk = chunk // B
    sl16 = NdP // NS

    scratch = [
        pltpu.VMEM((B,), jnp.int32),
        pltpu.VMEM((B,), jnp.float32),
        pltpu.VMEM((B,), jnp.float32),
        pltpu.VMEM((sl16,), jnp.float32),
        pltpu.VMEM_SHARED((NdP,), jnp.float32),
    ]
    if use_max:
        scratch.append(pltpu.VMEM((NdP,), jnp.float32))

    out_type = (jax.ShapeDtypeStruct((EP,), jnp.float32),
                jax.ShapeDtypeStruct((NC, NdP), jnp.float32))

    @functools.partial(pl.kernel, out_type=out_type, mesh=_MESH,
                       scratch_types=scratch, compiler_params=_SC_PARAMS)
    def k3(dst_hbm, l_hbm, *rest):
        if use_max:
            mfin_hbm, e_hbm, spart_hbm, dst_v, l_v, e_v, z_v, s_sh, m_v = rest
        else:
            e_hbm, spart_hbm, dst_v, l_v, e_v, z_v, s_sh = rest
        c = lax.axis_index("c")
        s = lax.axis_index("s")
        w = s * NC + c

        def zb(i, carry):
            z_v[pl.ds(i * LANES, LANES)] = jnp.zeros((LANES,), jnp.float32)
            return carry
        lax.fori_loop(0, sl16 // LANES, zb, 0)
        pltpu.sync_copy(z_v, s_sh.at[pl.ds(s * sl16, sl16)])
        if use_max:
            pltpu.sync_copy(mfin_hbm, m_v)
        plsc.subcore_barrier()

        def blk(b, carry):
            off = w * chunk + b * B
            pltpu.sync_copy(dst_hbm.at[pl.ds(off, B)], dst_v)
            pltpu.sync_copy(l_hbm.at[pl.ds(off, B)], l_v)

            def grp(g, gcarry):
                sl = pl.ds(g * LANES, LANES)
                l16 = l_v[sl]
                if use_max:
                    m16 = plsc.load_gather(m_v, [dst_v[sl]])
                    e_v[sl] = jnp.exp(l16 - m16)
                else:
                    e_v[sl] = jnp.exp(l16)
                return gcarry
            lax.fori_loop(0, B // LANES, grp, 0)
            pltpu.sync_copy(e_v, e_hbm.at[pl.ds(off, B)])
            pltpu.sync_copy(e_v, s_sh.at[dst_v], add=True)
            return carry
        lax.fori_loop(0, nblk, blk, 0)

        plsc.subcore_barrier()
        pltpu.sync_copy(s_sh.at[pl.ds(s * sl16, sl16)],
                        spart_hbm.at[c, pl.ds(s * sl16, sl16)])
    return k3


@functools.lru_cache(maxsize=None)
def _k5_scatter(EP, NdP, DSUB):
    npass = D // DSUB
    chunk = EP // NW
    nblk = chunk // B
    rows16 = NdP // NS

    scratch = [
        pltpu.VMEM((B,), jnp.int32),
        pltpu.VMEM((B,), jnp.int32),
        pltpu.VMEM((B,), jnp.float32),
        pltpu.VMEM((B, DSUB), jnp.float32),
        pltpu.VMEM((B, DSUB), jnp.float32),
        pltpu.VMEM((NdP,), jnp.float32),
        pltpu.VMEM((NdP,), jnp.float32),
        pltpu.VMEM_SHARED((NdP, DSUB), jnp.float32),
    ]

    @functools.partial(
        pl.kernel,
        out_type=jax.ShapeDtypeStruct((npass * NC, NdP, DSUB), jnp.float32),
        mesh=_MESH, scratch_types=scratch, compiler_params=_SC_PARAMS)
    def k5(*args):
        vq = args[:npass]
        (src_hbm, dst_hbm, e_hbm, spart_hbm, out_hbm,
         src_v, dst_v, e_v, vr_v, sc_v, s_v, t_v, slab) = args[npass:]
        c = lax.axis_index("c")
        s = lax.axis_index("s")
        w = s * NC + c

        pltpu.sync_copy(spart_hbm.at[0], s_v)
        pltpu.sync_copy(spart_hbm.at[1], t_v)

        def addch(i, carry):
            sl = pl.ds(i * LANES, LANES)
            s_v[sl] = s_v[sl] + t_v[sl]
            return carry
        lax.fori_loop(0, NdP // LANES, addch, 0)

        def zrow(i, carry):
            for c16 in range(DSUB // LANES):
                sc_v[i, pl.ds(c16 * LANES, LANES)] = jnp.zeros(
                    (LANES,), jnp.float32)
            return carry

        for p in range(npass):
            lax.fori_loop(0, B, zrow, 0)
            for q0 in range(0, rows16, B):
                n = min(B, rows16 - q0)
                pltpu.sync_copy(sc_v.at[pl.ds(0, n)],
                                slab.at[pl.ds(s * rows16 + q0, n)])
            plsc.subcore_barrier()

            def blk(b, carry):
                off = w * chunk + b * B
                pltpu.sync_copy(src_hbm.at[pl.ds(off, B)], src_v)
                pltpu.sync_copy(dst_hbm.at[pl.ds(off, B)], dst_v)
                pltpu.sync_copy(e_hbm.at[pl.ds(off, B)], e_v)
                pltpu.sync_copy(vq[p].at[src_v], vr_v)

                def grp(g, gcarry):
                    lane = _lane_iota()
                    sl = pl.ds(g * LANES, LANES)
                    d16 = dst_v[sl]
                    sv = plsc.load_gather(s_v, [d16])
                    a16 = e_v[sl] / (sv + 1e-16)
                    for j in range(LANES):
                        e = g * LANES + j
                        va = _bcast_lane(a16, lane, j)
                        for c16 in range(DSUB // LANES):
                            slc = pl.ds(c16 * LANES, LANES)
                            sc_v[e, slc] = vr_v[e, slc] * va
                    return gcarry
                lax.fori_loop(0, B // LANES, grp, 0)

                pltpu.sync_copy(sc_v, slab.at[dst_v], add=True)
                return carry
            lax.fori_loop(0, nblk, blk, 0)

            plsc.subcore_barrier()
            out_row = p * NC + c
            for q0 in range(0, rows16, B):
                n = min(B, rows16 - q0)
                pltpu.sync_copy(slab.at[pl.ds(s * rows16 + q0, n)],
                                out_hbm.at[out_row,
                                           pl.ds(s * rows16 + q0, n)])
            plsc.subcore_barrier()
    return k5


def _add_body(a_ref, b_ref, o_ref):
    o_ref[...] = a_ref[...] + b_ref[...]


def _pallas_add(a, b, block_rows=1024):
    r, d = a.shape
    grid = (r // block_rows,)
    return pl.pallas_call(
        _add_body,
        grid=grid,
        in_specs=[pl.BlockSpec((block_rows, d), lambda i: (i, 0)),
                  pl.BlockSpec((block_rows, d), lambda i: (i, 0))],
        out_specs=pl.BlockSpec((block_rows, d), lambda i: (i, 0)),
        out_shape=jax.ShapeDtypeStruct((r, d), jnp.float32),
    )(a, b)


def _round_up(x, m):
    return (x + m - 1) // m * m


_LEVEL = "full"


def _seg_softmax_jnp(logits, index, num_segments, use_max):
    if use_max:
        mx = jax.ops.segment_max(logits, index, num_segments=num_segments)
        mx = jnp.where(jnp.isfinite(mx), mx, 0.0)
        e = jnp.exp(logits - mx[index])
    else:
        e = jnp.exp(logits)
    s = jax.ops.segment_sum(e, index, num_segments=num_segments)
    return e / (s[index] + 1e-16)


def _sparse_attn_sc(q, k, v, src, dst, use_max):
    NdP = q.shape[0]
    EP = src.shape[0]
    if _LEVEL == "l":
        l = _k1_logits(EP, NdP, False)(q, k, src, dst)
        alpha = _seg_softmax_jnp(l, dst, NdP, use_max)
        return jax.ops.segment_sum(alpha[:, None] * v[src],
                                   dst, num_segments=NdP)
    if _LEVEL == "e":
        if use_max:
            l, mpart = _k1_logits(EP, NdP, True)(q, k, src, dst)
            mfin = _k2_merge(NdP)(mpart)
            e, spart = _k3_expsum(EP, NdP, True)(dst, l, mfin)
        else:
            l = _k1_logits(EP, NdP, False)(q, k, src, dst)
            e, spart = _k3_expsum(EP, NdP, False)(dst, l)
        s = spart[0] + spart[1]
        alpha = e / (s[dst] + 1e-16)
        return jax.ops.segment_sum(alpha[:, None] * v[src],
                                   dst, num_segments=NdP)
    if use_max:
        l, mpart = _k1_logits(EP, NdP, True)(q, k, src, dst)
        mfin = _k2_merge(NdP)(mpart)
        e, spart = _k3_expsum(EP, NdP, True)(dst, l, mfin)
    else:
        l = _k1_logits(EP, NdP, False)(q, k, src, dst)
        e, spart = _k3_expsum(EP, NdP, False)(dst, l)
    dsub = 64 if NdP <= 12288 else 32
    npass = D // dsub
    vq = [v[:, i * dsub:(i + 1) * dsub] for i in range(npass)]
    outp = _k5_scatter(EP, NdP, dsub)(*vq, src, dst, e, spart)
    a = jnp.transpose(outp[0::NC], (1, 0, 2)).reshape(NdP, D)
    b = jnp.transpose(outp[1::NC], (1, 0, 2)).reshape(NdP, D)
    return _pallas_add(a, b)


def _pad_edges(src, dst, ns, nd):
    E = src.shape[0]
    EP = _round_up(E, NW * B)
    src = jnp.pad(src.astype(jnp.int32), (0, EP - E), constant_values=ns)
    dst = jnp.pad(dst.astype(jnp.int32), (0, EP - E), constant_values=nd)
    return src, dst


def kernel(x_node, x_tri, params, node_edge_index, tri_edge_index,
           nt_edge_index, tn_edge_index):
    t, n, d = x_node.shape
    _, m, _ = x_tri.shape
    assert t == 1 and d == D
    NP = _round_up(n, 1024)
    MP = _round_up(m, 1024)
    scale = d ** (-0.5)

    xn = jnp.pad(x_node.reshape(n, d), ((0, NP - n), (0, 0)))
    xt = jnp.pad(x_tri.reshape(m, d), ((0, MP - m), (0, 0)))

    ne_src, ne_dst = _pad_edges(node_edge_index[0], node_edge_index[1], n, n)
    te_src, te_dst = _pad_edges(tri_edge_index[0], tri_edge_index[1], m, m)
    nt_src, nt_dst = _pad_edges(nt_edge_index[0], nt_edge_index[1], n, m)
    tn_src, tn_dst = _pad_edges(tn_edge_index[0], tn_edge_index[1], m, n)

    p = params['nsa']
    q, k, v = _qkv(xn, p, scale)
    h_node = _sparse_attn_sc(q, k, v, ne_src, ne_dst, False)

    p = params['n2t']
    q = _matmul_bias(xt, p['Wq'] * scale, p['bq'] * scale)
    kv = _matmul_bias(h_node,
                      jnp.concatenate([p['Wk'], p['Wv']], axis=1),
                      jnp.concatenate([p['bk'], p['bv']], axis=0))
    aggr = _sparse_attn_sc(q, kv[:, :D], kv[:, D:], nt_src, nt_dst, False)
    h_tri = _matmul_bias(aggr, p['Wo'], p['bo'], res=xt)

    p = params['tsa']
    q, k, v = _qkv(h_tri, p, d ** 0.5)
    h_tri2 = _sparse_attn_sc(q, k, v, te_src, te_dst, True)

    p = params['t2n']
    q = _matmul_bias(h_node, p['Wq'] * scale, p['bq'] * scale)
    kv = _matmul_bias(h_tri2,
                      jnp.concatenate([p['Wk'], p['Wv']], axis=1),
                      jnp.concatenate([p['bk'], p['bv']], axis=0))
    aggr = _sparse_attn_sc(q, kv[:, :D], kv[:, D:], tn_src, tn_dst, False)
    out_node = _matmul_bias(aggr, p['Wo'], p['bo'], res=h_node)
    return out_node[:n].reshape(t, n, d)


# speedup vs baseline: 1.7472x; 1.2815x over previous
import functools

import jax
import jax.numpy as jnp
import numpy as np
from jax import lax
from jax.experimental import pallas as pl
from jax.experimental.pallas import tpu as pltpu
from jax.experimental.pallas import tpu_sc as plsc

NC, NS, LANES = 2, 16, 16
NW = NC * NS
B = 128
D = 256
NEG = -3.0e38

_MESH = plsc.VectorSubcoreMesh(core_axis_name="c", subcore_axis_name="s")
_SC_PARAMS = pltpu.CompilerParams(needs_layout_passes=False,
                                  use_tc_tiling_on_sc=False)

def _lane_iota():
    return lax.broadcasted_iota(jnp.int32, (LANES,), 0)


def _lane_gather(vec16, idx16):
    dn = lax.GatherDimensionNumbers(offset_dims=(), collapsed_slice_dims=(0,),
                                    start_index_map=(0,))
    return lax.gather(vec16, idx16.reshape(LANES, 1), dn, slice_sizes=(1,),
                      mode=lax.GatherScatterMode.PROMISE_IN_BOUNDS)


def _bcast_lane(vec16, lane, j):
    return _lane_gather(vec16, lane * 0 + j)


def _lane_sum(acc, lane):
    for sh in (1, 2, 4, 8):
        acc = acc + _lane_gather(acc, lane ^ sh)
    return acc


def _mm_body(x_ref, w_ref, b_ref, o_ref):
    o_ref[...] = (
        jnp.dot(x_ref[...], w_ref[...], preferred_element_type=jnp.float32)
        + b_ref[...]
    )


def _mm_res_body(x_ref, w_ref, b_ref, r_ref, o_ref):
    o_ref[...] = (
        jnp.dot(x_ref[...], w_ref[...], preferred_element_type=jnp.float32)
        + b_ref[...] + r_ref[...]
    )


def _matmul_bias(x, w, b, res=None, block_rows=1024):
    r, d = x.shape
    dout = w.shape[1]
    assert r % block_rows == 0, (r, block_rows)
    grid = (r // block_rows,)
    if res is None:
        return pl.pallas_call(
            _mm_body,
            grid=grid,
            in_specs=[
                pl.BlockSpec((block_rows, d), lambda i: (i, 0)),
                pl.BlockSpec((d, dout), lambda i: (0, 0)),
                pl.BlockSpec((1, dout), lambda i: (0, 0)),
            ],
            out_specs=pl.BlockSpec((block_rows, dout), lambda i: (i, 0)),
            out_shape=jax.ShapeDtypeStruct((r, dout), jnp.float32),
        )(x, w, b.reshape(1, dout))
    return pl.pallas_call(
        _mm_res_body,
        grid=grid,
        in_specs=[
            pl.BlockSpec((block_rows, d), lambda i: (i, 0)),
            pl.BlockSpec((d, dout), lambda i: (0, 0)),
            pl.BlockSpec((1, dout), lambda i: (0, 0)),
            pl.BlockSpec((block_rows, dout), lambda i: (i, 0)),
        ],
        out_specs=pl.BlockSpec((block_rows, dout), lambda i: (i, 0)),
        out_shape=jax.ShapeDtypeStruct((r, dout), jnp.float32),
    )(x, w, b.reshape(1, dout), res)


def _qkv(x, p, scale):
    w = jnp.concatenate([p['Wq'] * scale, p['Wk'], p['Wv']], axis=1)
    b = jnp.concatenate([p['bq'] * scale, p['bk'], p['bv']], axis=0)
    out = _matmul_bias(x, w, b)
    return out[:, :D], out[:, D:2 * D], out[:, 2 * D:]


@functools.lru_cache(maxsize=None)
def _k1_logits(EP, NdP, use_max):
    chunk = EP // NW
    nblk = chunk // B

    scratch = [
        pltpu.VMEM((B,), jnp.int32),
        pltpu.VMEM((B,), jnp.int32),
        pltpu.VMEM((B, D), jnp.float32),
        pltpu.VMEM((B, D), jnp.float32),
        pltpu.VMEM((B,), jnp.float32),
    ]
    if use_max:
        scratch.append(pltpu.VMEM((NdP,), jnp.float32))
        out_type = (jax.ShapeDtypeStruct((EP,), jnp.float32),
                    jax.ShapeDtypeStruct((NW, NdP), jnp.float32))
    else:
        out_type = jax.ShapeDtypeStruct((EP,), jnp.float32)

    @functools.partial(pl.kernel, out_type=out_type, mesh=_MESH,
                       scratch_types=scratch, compiler_params=_SC_PARAMS)
    def k1(q_hbm, k_hbm, src_hbm, dst_hbm, *rest):
        if use_max:
            l_hbm, mpart_hbm, src_v, dst_v, q_v, k_v, l_v, m_v = rest
        else:
            l_hbm, src_v, dst_v, q_v, k_v, l_v = rest
        w = lax.axis_index("s") * NC + lax.axis_index("c")
        base = w * chunk

        if use_max:
            def initm(i, carry):
                m_v[pl.ds(i * LANES, LANES)] = jnp.full((LANES,), NEG,
                                                        jnp.float32)
                return carry
            lax.fori_loop(0, NdP // LANES, initm, 0)

        def blk(b, carry):
            off = base + b * B
            pltpu.sync_copy(src_hbm.at[pl.ds(off, B)], src_v)
            pltpu.sync_copy(dst_hbm.at[pl.ds(off, B)], dst_v)
            pltpu.sync_copy(q_hbm.at[dst_v], q_v)
            pltpu.sync_copy(k_hbm.at[src_v], k_v)

            def dotgrp(g, ecarry):
                lane = _lane_iota()
                lvec = jnp.zeros((LANES,), jnp.float32)
                for j in range(LANES):
                    e = g * LANES + j
                    acc = q_v[e, pl.ds(0, LANES)] * k_v[e, pl.ds(0, LANES)]
                    for c in range(1, D // LANES):
                        acc = acc + (q_v[e, pl.ds(c * LANES, LANES)]
                                     * k_v[e, pl.ds(c * LANES, LANES)])
                    lvec = jnp.where(lane == j, _lane_sum(acc, lane), lvec)
                l_v[pl.ds(g * LANES, LANES)] = lvec
                return ecarry
            lax.fori_loop(0, B // LANES, dotgrp, 0)

            if use_max:
                def grp(g, gcarry):
                    sl = pl.ds(g * LANES, LANES)
                    l16 = l_v[sl]
                    d16 = dst_v[sl]

                    def cond(c_):
                        return c_

                    def body(c_):
                        mo = plsc.load_gather(m_v, [d16])
                        plsc.store_scatter(m_v, [d16], l16, mask=l16 > mo)
                        mo2 = plsc.load_gather(m_v, [d16])
                        return jnp.any(l16 > mo2)
                    lax.while_loop(cond, body, True)
                    return gcarry
                lax.fori_loop(0, B // LANES, grp, 0)

            pltpu.sync_copy(l_v, l_hbm.at[pl.ds(off, B)])
            return carry
        lax.fori_loop(0, nblk, blk, 0)

        if use_max:
            pltpu.sync_copy(m_v, mpart_hbm.at[w])
    return k1


@functools.lru_cache(maxsize=None)
def _k2_merge(NdP):
    sl_len = NdP // NW

    @functools.partial(
        pl.kernel,
        out_type=jax.ShapeDtypeStruct((NdP,), jnp.float32),
        mesh=_MESH,
        scratch_types=[pltpu.VMEM((sl_len,), jnp.float32),
                       pltpu.VMEM((sl_len,), jnp.float32)],
        compiler_params=_SC_PARAMS)
    def k2(mpart_hbm, mfin_hbm, acc_v, tmp_v):
        w = lax.axis_index("s") * NC + lax.axis_index("c")
        off = w * sl_len
        pltpu.sync_copy(mpart_hbm.at[0, pl.ds(off, sl_len)], acc_v)

        def red(w2, carry):
            pltpu.sync_copy(mpart_hbm.at[w2, pl.ds(off, sl_len)], tmp_v)

            def ch(i, icarry):
                s_ = pl.ds(i * LANES, LANES)
                acc_v[s_] = jnp.maximum(acc_v[s_], tmp_v[s_])
                return icarry
            lax.fori_loop(0, sl_len // LANES, ch, 0)
            return carry
        lax.fori_loop(1, NW, red, 0)
        pltpu.sync_copy(acc_v, mfin_hbm.at[pl.ds(off, sl_len)])
    return k2


@functools.lru_cache(maxsize=None)
def _k3_expsum(EP, NdP, use_max):
    chunk = EP // NW
    nblk = chunk // B
    sl16 = NdP // NS

    scratch = [
        pltpu.VMEM((B,), jnp.int32),
        pltpu.VMEM((B,), jnp.float32),
        pltpu.VMEM((B,), jnp.float32),
        pltpu.VMEM((sl16,), jnp.float32),
        pltpu.VMEM_SHARED((NdP,), jnp.float32),
    ]
    if use_max:
        scratch.append(pltpu.VMEM((NdP,), jnp.float32))

    out_type = (jax.ShapeDtypeStruct((EP,), jnp.float32),
                jax.ShapeDtypeStruct((NC, NdP), jnp.float32))

    @functools.partial(pl.kernel, out_type=out_type, mesh=_MESH,
                       scratch_types=scratch, compiler_params=_SC_PARAMS)
    def k3(dst_hbm, l_hbm, *rest):
        if use_max:
            mfin_hbm, e_hbm, spart_hbm, dst_v, l_v, e_v, z_v, s_sh, m_v = rest
        else:
            e_hbm, spart_hbm, dst_v, l_v, e_v, z_v, s_sh = rest
        c = lax.axis_index("c")
        s = lax.axis_index("s")
        w = s * NC + c

        def zb(i, carry):
            z_v[pl.ds(i * LANES, LANES)] = jnp.zeros((LANES,), jnp.float32)
            return carry
        lax.fori_loop(0, sl16 // LANES, zb, 0)
        pltpu.sync_copy(z_v, s_sh.at[pl.ds(s * sl16, sl16)])
        if use_max:
            pltpu.sync_copy(mfin_hbm, m_v)
        plsc.subcore_barrier()

        def blk(b, carry):
            off = w * chunk + b * B
            pltpu.sync_copy(dst_hbm.at[pl.ds(off, B)], dst_v)
            pltpu.sync_copy(l_hbm.at[pl.ds(off, B)], l_v)

            def grp(g, gcarry):
                sl = pl.ds(g * LANES, LANES)
                l16 = l_v[sl]
                if use_max:
                    m16 = plsc.load_gather(m_v, [dst_v[sl]])
                    e_v[sl] = jnp.exp(l16 - m16)
                else:
                    e_v[sl] = jnp.exp(l16)
                return gcarry
            lax.fori_loop(0, B // LANES, grp, 0)
            pltpu.sync_copy(e_v, e_hbm.at[pl.ds(off, B)])
            pltpu.sync_copy(e_v, s_sh.at[dst_v], add=True)
            return carry
        lax.fori_loop(0, nblk, blk, 0)

        plsc.subcore_barrier()
        pltpu.sync_copy(s_sh.at[pl.ds(s * sl16, sl16)],
                        spart_hbm.at[c, pl.ds(s * sl16, sl16)])
    return k3


@functools.lru_cache(maxsize=None)
def _k5_scatter(EP, NdP, DSUB):
    npass = D // DSUB
    chunk = EP // NW
    nblk = chunk // B
    rows16 = NdP // NS

    scratch = [
        pltpu.VMEM((B,), jnp.int32),
        pltpu.VMEM((B,), jnp.int32),
        pltpu.VMEM((B,), jnp.float32),
        pltpu.VMEM((B, DSUB), jnp.float32),
        pltpu.VMEM((B, DSUB), jnp.float32),
        pltpu.VMEM((NdP,), jnp.float32),
        pltpu.VMEM_SHARED((NdP, DSUB), jnp.float32),
    ]

    @functools.partial(
        pl.kernel,
        out_type=jax.ShapeDtypeStruct((npass * NC, NdP, DSUB), jnp.float32),
        mesh=_MESH, scratch_types=scratch, compiler_params=_SC_PARAMS)
    def k5(*args):
        vq = args[:npass]
        (src_hbm, dst_hbm, e_hbm, s_hbm, out_hbm,
         src_v, dst_v, e_v, vr_v, sc_v, s_v, slab) = args[npass:]
        c = lax.axis_index("c")
        s = lax.axis_index("s")
        w = s * NC + c

        pltpu.sync_copy(s_hbm, s_v)

        def zrow(i, carry):
            for c16 in range(DSUB // LANES):
                sc_v[i, pl.ds(c16 * LANES, LANES)] = jnp.zeros(
                    (LANES,), jnp.float32)
            return carry

        for p in range(npass):
            lax.fori_loop(0, B, zrow, 0)
            for q0 in range(0, rows16, B):
                n = min(B, rows16 - q0)
                pltpu.sync_copy(sc_v.at[pl.ds(0, n)],
                                slab.at[pl.ds(s * rows16 + q0, n)])
            plsc.subcore_barrier()

            def blk(b, carry):
                off = w * chunk + b * B
                pltpu.sync_copy(src_hbm.at[pl.ds(off, B)], src_v)
                pltpu.sync_copy(dst_hbm.at[pl.ds(off, B)], dst_v)
                pltpu.sync_copy(e_hbm.at[pl.ds(off, B)], e_v)
                pltpu.sync_copy(vq[p].at[src_v], vr_v)

                def grp(g, gcarry):
                    lane = _lane_iota()
                    sl = pl.ds(g * LANES, LANES)
                    d16 = dst_v[sl]
                    sv = plsc.load_gather(s_v, [d16])
                    a16 = e_v[sl] / (sv + 1e-16)
                    for j in range(LANES):
                        e = g * LANES + j
                        va = _bcast_lane(a16, lane, j)
                        for c16 in range(DSUB // LANES):
                            slc = pl.ds(c16 * LANES, LANES)
                            sc_v[e, slc] = vr_v[e, slc] * va
                    return gcarry
                lax.fori_loop(0, B // LANES, grp, 0)

                pltpu.sync_copy(sc_v, slab.at[dst_v], add=True)
                return carry
            lax.fori_loop(0, nblk, blk, 0)

            plsc.subcore_barrier()
            out_row = p * NC + c
            for q0 in range(0, rows16, B):
                n = min(B, rows16 - q0)
                pltpu.sync_copy(slab.at[pl.ds(s * rows16 + q0, n)],
                                out_hbm.at[out_row,
                                           pl.ds(s * rows16 + q0, n)])
            plsc.subcore_barrier()
    return k5


def _add_body(a_ref, b_ref, o_ref):
    o_ref[...] = a_ref[...] + b_ref[...]


def _pallas_add(a, b, block_rows=1024):
    r, d = a.shape
    grid = (r // block_rows,)
    return pl.pallas_call(
        _add_body,
        grid=grid,
        in_specs=[pl.BlockSpec((block_rows, d), lambda i: (i, 0)),
                  pl.BlockSpec((block_rows, d), lambda i: (i, 0))],
        out_specs=pl.BlockSpec((block_rows, d), lambda i: (i, 0)),
        out_shape=jax.ShapeDtypeStruct((r, d), jnp.float32),
    )(a, b)


def _round_up(x, m):
    return (x + m - 1) // m * m


_LEVEL = "full"


def _seg_softmax_jnp(logits, index, num_segments, use_max):
    if use_max:
        mx = jax.ops.segment_max(logits, index, num_segments=num_segments)
        mx = jnp.where(jnp.isfinite(mx), mx, 0.0)
        e = jnp.exp(logits - mx[index])
    else:
        e = jnp.exp(logits)
    s = jax.ops.segment_sum(e, index, num_segments=num_segments)
    return e / (s[index] + 1e-16)


def _sparse_attn_sc(q, k, v, src, dst, use_max):
    NdP = q.shape[0]
    EP = src.shape[0]
    if _LEVEL == "l":
        l = _k1_logits(EP, NdP, False)(q, k, src, dst)
        alpha = _seg_softmax_jnp(l, dst, NdP, use_max)
        return jax.ops.segment_sum(alpha[:, None] * v[src],
                                   dst, num_segments=NdP)
    if _LEVEL == "e":
        if use_max:
            l, mpart = _k1_logits(EP, NdP, True)(q, k, src, dst)
            mfin = _k2_merge(NdP)(mpart)
            e, spart = _k3_expsum(EP, NdP, True)(dst, l, mfin)
        else:
            l = _k1_logits(EP, NdP, False)(q, k, src, dst)
            e, spart = _k3_expsum(EP, NdP, False)(dst, l)
        s = spart[0] + spart[1]
        alpha = e / (s[dst] + 1e-16)
        return jax.ops.segment_sum(alpha[:, None] * v[src],
                                   dst, num_segments=NdP)
    if use_max:
        l, mpart = _k1_logits(EP, NdP, True)(q, k, src, dst)
        mfin = _k2_merge(NdP)(mpart)
        e, spart = _k3_expsum(EP, NdP, True)(dst, l, mfin)
    else:
        l = _k1_logits(EP, NdP, False)(q, k, src, dst)
        e, spart = _k3_expsum(EP, NdP, False)(dst, l)
    dsub = 128 if NdP <= 12288 else 64
    npass = D // dsub
    vq = [v[:, i * dsub:(i + 1) * dsub] for i in range(npass)]
    srows = NdP // 128
    s = _pallas_add(spart[0].reshape(srows, 128),
                    spart[1].reshape(srows, 128),
                    block_rows=srows).reshape(NdP)
    outp = _k5_scatter(EP, NdP, dsub)(*vq, src, dst, e, s)
    a = jnp.transpose(outp[0::NC], (1, 0, 2)).reshape(NdP, D)
    b = jnp.transpose(outp[1::NC], (1, 0, 2)).reshape(NdP, D)
    return _pallas_add(a, b)


def _pad_edges(src, dst, ns, nd):
    E = src.shape[0]
    EP = _round_up(E, NW * B)
    src = jnp.pad(src.astype(jnp.int32), (0, EP - E), constant_values=ns)
    dst = jnp.pad(dst.astype(jnp.int32), (0, EP - E), constant_values=nd)
    return src, dst


def kernel(x_node, x_tri, params, node_edge_index, tri_edge_index,
           nt_edge_index, tn_edge_index):
    t, n, d = x_node.shape
    _, m, _ = x_tri.shape
    assert t == 1 and d == D
    NP = _round_up(n, 1024)
    MP = _round_up(m, 1024)
    scale = d ** (-0.5)

    xn = jnp.pad(x_node.reshape(n, d), ((0, NP - n), (0, 0)))
    xt = jnp.pad(x_tri.reshape(m, d), ((0, MP - m), (0, 0)))

    ne_src, ne_dst = _pad_edges(node_edge_index[0], node_edge_index[1], n, n)
    te_src, te_dst = _pad_edges(tri_edge_index[0], tri_edge_index[1], m, m)
    nt_src, nt_dst = _pad_edges(nt_edge_index[0], nt_edge_index[1], n, m)
    tn_src, tn_dst = _pad_edges(tn_edge_index[0], tn_edge_index[1], m, n)

    p = params['nsa']
    q, k, v = _qkv(xn, p, scale)
    h_node = _sparse_attn_sc(q, k, v, ne_src, ne_dst, False)

    p = params['n2t']
    q = _matmul_bias(xt, p['Wq'] * scale, p['bq'] * scale)
    kv = _matmul_bias(h_node,
                      jnp.concatenate([p['Wk'], p['Wv']], axis=1),
                      jnp.concatenate([p['bk'], p['bv']], axis=0))
    aggr = _sparse_attn_sc(q, kv[:, :D], kv[:, D:], nt_src, nt_dst, False)
    h_tri = _matmul_bias(aggr, p['Wo'], p['bo'], res=xt)

    p = params['tsa']
    q, k, v = _qkv(h_tri, p, d ** 0.5)
    h_tri2 = _sparse_attn_sc(q, k, v, te_src, te_dst, True)

    p = params['t2n']
    q = _matmul_bias(h_node, p['Wq'] * scale, p['bq'] * scale)
    kv = _matmul_bias(h_tri2,
                      jnp.concatenate([p['Wk'], p['Wv']], axis=1),
                      jnp.concatenate([p['bk'], p['bv']], axis=0))
    aggr = _sparse_attn_sc(q, kv[:, :D], kv[:, D:], tn_src, tn_dst, False)
    out_node = _matmul_bias(aggr, p['Wo'], p['bo'], res=h_node)
    return out_node[:n].reshape(t, n, d)
